# baseline (speedup 1.0000x reference)
import jax
import jax.numpy as jnp
import numpy as np
from jax.experimental import pallas as pl
from jax.experimental.pallas import tpu as pltpu

STRIDE = 16
ANCHOR_SIZE = 128.0
ASPECT_RATIOS = (0.2323283, 0.63365731, 1.28478321, 3.15089189)
IMG_H, IMG_W = 800, 1216
PRE_NMS_TOP_N = 6000
POST_NMS_TOP_N = 1000
OUT_PAD_ROWS = 1152
NMS_THRESH = 0.7
BBOX_XFORM_CLIP = float(np.log(1000.0 / 16.0))
H_FEAT, W_FEAT = 50, 76
A = 4
NPAD = 6144
NROWS = NPAD // 128


def _cell_anchors():
    out = []
    for r in ASPECT_RATIOS:
        w = np.round(np.sqrt(ANCHOR_SIZE * ANCHOR_SIZE / r))
        h = np.round(w * r)
        xc = yc = (STRIDE - 1.0) / 2.0
        out.append([xc - 0.5 * (w - 1), yc - 0.5 * (h - 1), xc + 0.5 * (w - 1), yc + 0.5 * (h - 1)])
    return jnp.asarray(out, dtype=jnp.float32)


def _grid_anchors(H, W):
    base = _cell_anchors()
    sx = jnp.arange(W, dtype=jnp.float32) * STRIDE
    sy = jnp.arange(H, dtype=jnp.float32) * STRIDE
    gy, gx = jnp.meshgrid(sy, sx, indexing='ij')
    shifts = jnp.stack([gx.ravel(), gy.ravel(), gx.ravel(), gy.ravel()], axis=1)
    return (shifts[:, None, :] + base[None, :, :]).reshape(-1, 4)


def _conv_same(x, w, b):
    y = jax.lax.conv_general_dilated(x, w, (1, 1), 'SAME', dimension_numbers=('NCHW', 'OIHW', 'NCHW'))
    return y + b[None, :, None, None]


def _nms_body(s_ref, dx_ref, dy_ref, dw_ref, dh_ref,
              ax1_ref, ay1_ref, ax2_ref, ay2_ref, out_ref,
              sw_ref, x1_ref, y1_ref, x2_ref, y2_ref, ar_ref, sc_ref):
    s0 = s_ref[...]
    ax1 = ax1_ref[...]
    ay1 = ay1_ref[...]
    ax2 = ax2_ref[...]
    ay2 = ay2_ref[...]
    tw = ax2 - ax1 + 1.0
    th = ay2 - ay1 + 1.0
    cx = ax1 + 0.5 * tw
    cy = ay1 + 0.5 * th
    dw = jnp.minimum(dw_ref[...], BBOX_XFORM_CLIP)
    dh = jnp.minimum(dh_ref[...], BBOX_XFORM_CLIP)
    px = dx_ref[...] * tw + cx
    py = dy_ref[...] * th + cy
    pw = jnp.exp(dw) * tw
    ph = jnp.exp(dh) * th
    x1 = jnp.clip(px - 0.5 * pw, 0.0, IMG_W - 1.0)
    y1 = jnp.clip(py - 0.5 * ph, 0.0, IMG_H - 1.0)
    x2 = jnp.clip(px + 0.5 * pw - 1.0, 0.0, IMG_W - 1.0)
    y2 = jnp.clip(py + 0.5 * ph - 1.0, 0.0, IMG_H - 1.0)
    areas = (x2 - x1 + 1.0) * (y2 - y1 + 1.0)
    sw_ref[...] = s0
    x1_ref[...] = x1
    y1_ref[...] = y1
    x2_ref[...] = x2
    y2_ref[...] = y2
    ar_ref[...] = areas
    sc_ref[...] = s0

    lane = jax.lax.broadcasted_iota(jnp.int32, (1, 128), 1)
    neg_inf = jnp.float32(-jnp.inf)

    sub128 = jax.lax.broadcasted_iota(jnp.int32, (128, 128), 0)
    lane_sq = jax.lax.broadcasted_iota(jnp.int32, (128, 128), 1)
    k_lt_e = sub128 < lane_sq
    one_sq = jnp.float32(1.0)

    def row_bcast(ref, c):
        return jnp.broadcast_to(ref[pl.ds(c, 1), :], (128, 128))

    def transpose_sq(sq):
        return jnp.transpose(sq, (1, 0))

    def iou_tile(kx1, ky1, kx2, ky2, kar, c_e):
        nr = kx1.shape[0]

        def rb(ref):
            return jnp.broadcast_to(ref[pl.ds(c_e, 1), :], (nr, 128))

        xx1 = jnp.maximum(kx1, rb(x1_ref))
        yy1 = jnp.maximum(ky1, rb(y1_ref))
        xx2 = jnp.minimum(kx2, rb(x2_ref))
        yy2 = jnp.minimum(ky2, rb(y2_ref))
        inter = jnp.maximum(0.0, xx2 - xx1 + 1.0) * jnp.maximum(0.0, yy2 - yy1 + 1.0)
        return inter / (kar + rb(ar_ref) - inter)

    def counts_of(kept_f, o_tile):
        return jax.lax.dot_general(kept_f, o_tile, (((1,), (0,)), ((), ())),
                                   preferred_element_type=jnp.float32)

    row0 = jnp.where(lane == 0, x1_ref[0, 0],
           jnp.where(lane == 1, y1_ref[0, 0],
           jnp.where(lane == 2, x2_ref[0, 0],
           jnp.where(lane == 3, y2_ref[0, 0],
           jnp.where(lane == 4, sc_ref[0, 0], 0.0)))))
    row0_sq = jnp.broadcast_to(row0, (128, 128))
    for i in range(OUT_PAD_ROWS // 128):
        out_ref[pl.ds(i * 128, 128), :] = row0_sq

    def chunk_body(c, t):
        kx1 = transpose_sq(row_bcast(x1_ref, c))
        ky1 = transpose_sq(row_bcast(y1_ref, c))
        kx2 = transpose_sq(row_bcast(x2_ref, c))
        ky2 = transpose_sq(row_bcast(y2_ref, c))
        kar = transpose_sq(row_bcast(ar_ref, c))
        ksc = transpose_sq(row_bcast(sc_ref, c))

        alive = sw_ref[pl.ds(c, 1), :] > neg_inf
        self_o = jnp.where((iou_tile(kx1, ky1, kx2, ky2, kar, c) > NMS_THRESH) & k_lt_e,
                           1.0, 0.0)

        def fcond(s):
            return s[1]

        def fbody(s):
            kept, _ = s
            cnt = counts_of(kept, self_o)
            new = jnp.where(alive & (cnt == 0.0), 1.0, 0.0)
            return new, jnp.any(new != kept)

        kept, _ = jax.lax.while_loop(
            fcond, fbody, (jnp.where(alive, 1.0, 0.0), True))

        tot = jnp.sum(kept).astype(jnp.int32)
        prefix = counts_of(kept, jnp.where(k_lt_e, 1.0, 0.0))
        prefix_i = prefix.astype(jnp.int32)

        @pl.when(t < POST_NMS_TOP_N)
        def _write():
            onehot = jnp.where((jnp.broadcast_to(prefix_i, (128, 128))
                                == sub128)
                               & (jnp.broadcast_to(kept, (128, 128)) > 0.0), 1.0, 0.0)
            data = jnp.where(lane_sq == 0, kx1,
                   jnp.where(lane_sq == 1, ky1,
                   jnp.where(lane_sq == 2, kx2,
                   jnp.where(lane_sq == 3, ky2,
                   jnp.where(lane_sq == 4, ksc, 0.0)))))
            rows = jax.lax.dot_general(onehot, data, (((1,), (0,)), ((), ())),
                                       precision=jax.lax.Precision.HIGHEST,
                                       preferred_element_type=jnp.float32)
            junk = jax.lax.broadcasted_iota(jnp.int32, (128, 128), 0) >= tot
            rows = rows + jnp.where(junk, jnp.broadcast_to(row0, (128, 128)), 0.0)
            out_ref[pl.ds(t, 128), :] = rows

        sub64 = jax.lax.broadcasted_iota(jnp.int32, (64, 128), 0)
        ohc = jnp.where((jnp.broadcast_to(prefix_i, (64, 128)) == sub64)
                        & (jnp.broadcast_to(kept, (64, 128)) > 0.0), 1.0, 0.0)

        def compact(sq_t):
            return jax.lax.dot_general(ohc, sq_t, (((1,), (0,)), ((), ())),
                                       precision=jax.lax.Precision.HIGHEST,
                                       preferred_element_type=jnp.float32)

        kx1c = compact(kx1)
        ky1c = compact(ky1)
        kx2c = jnp.where(sub64 >= tot, jnp.float32(-1e9), compact(kx2))
        ky2c = compact(ky2)
        karc = compact(kar)
        ones64 = jnp.ones((1, 64), jnp.float32)

        @pl.when(jnp.logical_and(t < POST_NMS_TOP_N, tot <= 64))
        def _cross_small():
            def cross(cp, _):
                o = jnp.where(iou_tile(kx1c, ky1c, kx2c, ky2c, karc, cp)
                              > NMS_THRESH, 1.0, 0.0)
                cnt = jax.lax.dot_general(ones64, o, (((1,), (0,)), ((), ())),
                                          preferred_element_type=jnp.float32)
                sw_ref[pl.ds(cp, 1), :] = jnp.where(
                    cnt > 0.0, neg_inf, sw_ref[pl.ds(cp, 1), :])
                return 0

            jax.lax.fori_loop(c + 1, NROWS, cross, 0)

        @pl.when(jnp.logical_and(t < POST_NMS_TOP_N, tot > 64))
        def _cross_full():
            def cross(cp, _):
                o = jnp.where(iou_tile(kx1, ky1, kx2, ky2, kar, cp)
                              > NMS_THRESH, 1.0, 0.0)
                cnt = counts_of(kept, o)
                sw_ref[pl.ds(cp, 1), :] = jnp.where(
                    cnt > 0.0, neg_inf, sw_ref[pl.ds(cp, 1), :])
                return 0

            jax.lax.fori_loop(c + 1, NROWS, cross, 0)

        return t + tot

    jax.lax.fori_loop(0, NROWS, chunk_body, 0)


def _heads_body(t_ref, w_ref, b_ref, o_ref):
    h = jax.lax.dot_general(t_ref[...], w_ref[...], (((1,), (0,)), ((), ())),
                            preferred_element_type=jnp.float32) + b_ref[...]
    col = jax.lax.broadcasted_iota(jnp.int32, (3840, 128), 1)
    o_ref[...] = jnp.where(col < A, jax.nn.sigmoid(h), h)


def kernel(features, W_conv, b_conv, W_obj, b_obj, W_reg, b_reg):
    t = jax.nn.relu(_conv_same(features, W_conv, b_conv))
    H, W = H_FEAT, W_FEAT
    t_mat = jnp.pad(jnp.transpose(t[0], (1, 2, 0)).reshape(H * W, 256), ((0, 40), (0, 0)))
    w_heads = jnp.zeros((256, 128), jnp.float32)
    w_heads = w_heads.at[:, :A].set(W_obj[:, :, 0, 0].T)
    w_heads = w_heads.at[:, A:A + 4 * A].set(W_reg[:, :, 0, 0].T)
    b_heads = jnp.zeros((1, 128), jnp.float32)
    b_heads = b_heads.at[0, :A].set(b_obj)
    b_heads = b_heads.at[0, A:A + 4 * A].set(b_reg)
    heads = pl.pallas_call(
        _heads_body,
        out_shape=jax.ShapeDtypeStruct((3840, 128), jnp.float32),
    )(t_mat, w_heads, b_heads)
    scores = heads[:H * W, :A].reshape(-1)
    reg = heads[:H * W, A:A + 4 * A].reshape(1, -1, 4)
    anchors = _grid_anchors(H, W)
    top_scores, top_idx = jax.lax.top_k(scores, PRE_NMS_TOP_N)
    codes = reg[0][top_idx]
    anc = anchors[top_idx]

    def pad(v, fill):
        return jnp.full((NPAD,), fill, jnp.float32).at[:PRE_NMS_TOP_N].set(v).reshape(NROWS, 128)

    args = (
        pad(top_scores, -jnp.inf),
        pad(codes[:, 0], 0.0), pad(codes[:, 1], 0.0),
        pad(codes[:, 2], 0.0), pad(codes[:, 3], 0.0),
        pad(anc[:, 0], 0.0), pad(anc[:, 1], 0.0),
        pad(anc[:, 2], 15.0), pad(anc[:, 3], 15.0),
    )
    out = pl.pallas_call(
        _nms_body,
        out_shape=jax.ShapeDtypeStruct((OUT_PAD_ROWS, 128), jnp.float32),
        scratch_shapes=[pltpu.VMEM((NROWS, 128), jnp.float32)] * 7,
    )(*args)
    return out[:POST_NMS_TOP_N, :5]


# speedup vs baseline: 1.0433x; 1.0433x over previous
import jax
import jax.numpy as jnp
import numpy as np
from jax.experimental import pallas as pl
from jax.experimental.pallas import tpu as pltpu

STRIDE = 16
ANCHOR_SIZE = 128.0
ASPECT_RATIOS = (0.2323283, 0.63365731, 1.28478321, 3.15089189)
IMG_H, IMG_W = 800, 1216
PRE_NMS_TOP_N = 6000
POST_NMS_TOP_N = 1000
OUT_PAD_ROWS = 1152
NMS_THRESH = 0.7
BBOX_XFORM_CLIP = float(np.log(1000.0 / 16.0))
H_FEAT, W_FEAT = 50, 76
A = 4
NPAD = 6144
NROWS = NPAD // 128


def _cell_anchors():
    out = []
    for r in ASPECT_RATIOS:
        w = np.round(np.sqrt(ANCHOR_SIZE * ANCHOR_SIZE / r))
        h = np.round(w * r)
        xc = yc = (STRIDE - 1.0) / 2.0
        out.append([xc - 0.5 * (w - 1), yc - 0.5 * (h - 1), xc + 0.5 * (w - 1), yc + 0.5 * (h - 1)])
    return jnp.asarray(out, dtype=jnp.float32)


def _grid_anchors(H, W):
    base = _cell_anchors()
    sx = jnp.arange(W, dtype=jnp.float32) * STRIDE
    sy = jnp.arange(H, dtype=jnp.float32) * STRIDE
    gy, gx = jnp.meshgrid(sy, sx, indexing='ij')
    shifts = jnp.stack([gx.ravel(), gy.ravel(), gx.ravel(), gy.ravel()], axis=1)
    return (shifts[:, None, :] + base[None, :, :]).reshape(-1, 4)


def _conv_same(x, w, b):
    y = jax.lax.conv_general_dilated(x, w, (1, 1), 'SAME', dimension_numbers=('NCHW', 'OIHW', 'NCHW'))
    return y + b[None, :, None, None]


def _nms_body(s_ref, dx_ref, dy_ref, dw_ref, dh_ref,
              ax1_ref, ay1_ref, ax2_ref, ay2_ref, out_ref,
              sw_ref, x1_ref, y1_ref, x2_ref, y2_ref, ar_ref, sc_ref):
    s0 = s_ref[...]
    ax1 = ax1_ref[...]
    ay1 = ay1_ref[...]
    ax2 = ax2_ref[...]
    ay2 = ay2_ref[...]
    tw = ax2 - ax1 + 1.0
    th = ay2 - ay1 + 1.0
    cx = ax1 + 0.5 * tw
    cy = ay1 + 0.5 * th
    dw = jnp.minimum(dw_ref[...], BBOX_XFORM_CLIP)
    dh = jnp.minimum(dh_ref[...], BBOX_XFORM_CLIP)
    px = dx_ref[...] * tw + cx
    py = dy_ref[...] * th + cy
    pw = jnp.exp(dw) * tw
    ph = jnp.exp(dh) * th
    x1 = jnp.clip(px - 0.5 * pw, 0.0, IMG_W - 1.0)
    y1 = jnp.clip(py - 0.5 * ph, 0.0, IMG_H - 1.0)
    x2 = jnp.clip(px + 0.5 * pw - 1.0, 0.0, IMG_W - 1.0)
    y2 = jnp.clip(py + 0.5 * ph - 1.0, 0.0, IMG_H - 1.0)
    areas = (x2 - x1 + 1.0) * (y2 - y1 + 1.0)
    sw_ref[...] = s0
    x1_ref[...] = x1
    y1_ref[...] = y1
    x2_ref[...] = x2
    y2_ref[...] = y2
    ar_ref[...] = areas
    sc_ref[...] = s0

    lane = jax.lax.broadcasted_iota(jnp.int32, (1, 128), 1)
    neg_inf = jnp.float32(-jnp.inf)

    sub128 = jax.lax.broadcasted_iota(jnp.int32, (128, 128), 0)
    lane_sq = jax.lax.broadcasted_iota(jnp.int32, (128, 128), 1)
    k_lt_e = sub128 < lane_sq
    one_sq = jnp.float32(1.0)

    def row_bcast(ref, c):
        return jnp.broadcast_to(ref[pl.ds(c, 1), :], (128, 128))

    def transpose_sq(sq):
        return jnp.transpose(sq, (1, 0))

    def iou_tile(kx1, ky1, kx2, ky2, kar, c_e):
        nr = kx1.shape[0]

        def rb(ref):
            return jnp.broadcast_to(ref[pl.ds(c_e, 1), :], (nr, 128))

        xx1 = jnp.maximum(kx1, rb(x1_ref))
        yy1 = jnp.maximum(ky1, rb(y1_ref))
        xx2 = jnp.minimum(kx2, rb(x2_ref))
        yy2 = jnp.minimum(ky2, rb(y2_ref))
        inter = jnp.maximum(0.0, xx2 - xx1 + 1.0) * jnp.maximum(0.0, yy2 - yy1 + 1.0)
        return inter / (kar + rb(ar_ref) - inter)

    def counts_of(kept_f, o_tile):
        return jax.lax.dot_general(kept_f, o_tile, (((1,), (0,)), ((), ())),
                                   preferred_element_type=jnp.float32)

    row0 = jnp.where(lane == 0, x1_ref[0, 0],
           jnp.where(lane == 1, y1_ref[0, 0],
           jnp.where(lane == 2, x2_ref[0, 0],
           jnp.where(lane == 3, y2_ref[0, 0],
           jnp.where(lane == 4, sc_ref[0, 0], 0.0)))))
    row0_sq = jnp.broadcast_to(row0, (128, 128))
    for i in range(OUT_PAD_ROWS // 128):
        out_ref[pl.ds(i * 128, 128), :] = row0_sq

    def chunk_body(c, t):
        kx1 = transpose_sq(row_bcast(x1_ref, c))
        ky1 = transpose_sq(row_bcast(y1_ref, c))
        kx2 = transpose_sq(row_bcast(x2_ref, c))
        ky2 = transpose_sq(row_bcast(y2_ref, c))
        kar = transpose_sq(row_bcast(ar_ref, c))
        ksc = transpose_sq(row_bcast(sc_ref, c))

        alive = sw_ref[pl.ds(c, 1), :] > neg_inf
        self_o = jnp.where((iou_tile(kx1, ky1, kx2, ky2, kar, c) > NMS_THRESH) & k_lt_e,
                           1.0, 0.0)

        def fcond(s):
            return s[1]

        def fbody(s):
            kept, _ = s
            cnt = counts_of(kept, self_o)
            new = jnp.where(alive & (cnt == 0.0), 1.0, 0.0)
            return new, jnp.any(new != kept)

        kept, _ = jax.lax.while_loop(
            fcond, fbody, (jnp.where(alive, 1.0, 0.0), True))

        tot = jnp.sum(kept).astype(jnp.int32)
        prefix = counts_of(kept, jnp.where(k_lt_e, 1.0, 0.0))
        prefix_i = prefix.astype(jnp.int32)

        @pl.when(t < POST_NMS_TOP_N)
        def _write():
            onehot = jnp.where((jnp.broadcast_to(prefix_i, (128, 128))
                                == sub128)
                               & (jnp.broadcast_to(kept, (128, 128)) > 0.0), 1.0, 0.0)
            data = jnp.where(lane_sq == 0, kx1,
                   jnp.where(lane_sq == 1, ky1,
                   jnp.where(lane_sq == 2, kx2,
                   jnp.where(lane_sq == 3, ky2,
                   jnp.where(lane_sq == 4, ksc, 0.0)))))
            rows = jax.lax.dot_general(onehot, data, (((1,), (0,)), ((), ())),
                                       precision=jax.lax.Precision.HIGHEST,
                                       preferred_element_type=jnp.float32)
            junk = jax.lax.broadcasted_iota(jnp.int32, (128, 128), 0) >= tot
            rows = rows + jnp.where(junk, jnp.broadcast_to(row0, (128, 128)), 0.0)
            out_ref[pl.ds(t, 128), :] = rows

        def cross(cp, _):
            @pl.when(t < POST_NMS_TOP_N)
            def _():
                o = jnp.where(iou_tile(kx1, ky1, kx2, ky2, kar, cp) > NMS_THRESH,
                              1.0, 0.0)
                cnt = counts_of(kept, o)
                sw_ref[pl.ds(cp, 1), :] = jnp.where(
                    cnt > 0.0, neg_inf, sw_ref[pl.ds(cp, 1), :])
            return 0

        jax.lax.fori_loop(c + 1, NROWS, cross, 0)
        return t + tot

    jax.lax.fori_loop(0, NROWS, chunk_body, 0)


def _heads_body(t_ref, w_ref, b_ref, o_ref):
    h = jax.lax.dot_general(t_ref[...], w_ref[...], (((1,), (0,)), ((), ())),
                            preferred_element_type=jnp.float32) + b_ref[...]
    col = jax.lax.broadcasted_iota(jnp.int32, (3840, 128), 1)
    o_ref[...] = jnp.where(col < A, jax.nn.sigmoid(h), h)


def kernel(features, W_conv, b_conv, W_obj, b_obj, W_reg, b_reg):
    t = jax.nn.relu(_conv_same(features, W_conv, b_conv))
    H, W = H_FEAT, W_FEAT
    t_mat = jnp.pad(jnp.transpose(t[0], (1, 2, 0)).reshape(H * W, 256), ((0, 40), (0, 0)))
    w_heads = jnp.zeros((256, 128), jnp.float32)
    w_heads = w_heads.at[:, :A].set(W_obj[:, :, 0, 0].T)
    w_heads = w_heads.at[:, A:A + 4 * A].set(W_reg[:, :, 0, 0].T)
    b_heads = jnp.zeros((1, 128), jnp.float32)
    b_heads = b_heads.at[0, :A].set(b_obj)
    b_heads = b_heads.at[0, A:A + 4 * A].set(b_reg)
    heads = pl.pallas_call(
        _heads_body,
        out_shape=jax.ShapeDtypeStruct((3840, 128), jnp.float32),
    )(t_mat, w_heads, b_heads)
    scores = heads[:H * W, :A].reshape(-1)
    reg = heads[:H * W, A:A + 4 * A].reshape(1, -1, 4)
    anchors = _grid_anchors(H, W)
    top_scores, top_idx = jax.lax.top_k(scores, PRE_NMS_TOP_N)
    codes = reg[0][top_idx]
    anc = anchors[top_idx]

    def pad(v, fill):
        return jnp.full((NPAD,), fill, jnp.float32).at[:PRE_NMS_TOP_N].set(v).reshape(NROWS, 128)

    args = (
        pad(top_scores, -jnp.inf),
        pad(codes[:, 0], 0.0), pad(codes[:, 1], 0.0),
        pad(codes[:, 2], 0.0), pad(codes[:, 3], 0.0),
        pad(anc[:, 0], 0.0), pad(anc[:, 1], 0.0),
        pad(anc[:, 2], 15.0), pad(anc[:, 3], 15.0),
    )
    out = pl.pallas_call(
        _nms_body,
        out_shape=jax.ShapeDtypeStruct((OUT_PAD_ROWS, 128), jnp.float32),
        scratch_shapes=[pltpu.VMEM((NROWS, 128), jnp.float32)] * 7,
    )(*args)
    return out[:POST_NMS_TOP_N, :5]
